# Initial kernel scaffold; baseline (speedup 1.0000x reference)
import jax
import jax.numpy as jnp
from jax.experimental import pallas as pl


def kernel(labels, table):
    raise NotImplementedError("write your pallas kernel here")


# speedup vs baseline: 3.2126x; 3.2126x over previous
import functools

import jax
import jax.numpy as jnp
from jax import lax
from jax.experimental import pallas as pl
from jax.experimental.pallas import tpu as pltpu
from jax.experimental.pallas import tpu_sc as plsc

_NUM_CLASSES = 100000
_LATENT_DIM = 128
_BATCH = 16384
_FIELDS = 26

_NW = 32
_CHUNK = 416


def _gather_kernel(idx_hbm, table_hbm, out_hbm, idx_v, rows_v, sem):
    b_per_w = (_BATCH * _FIELDS) // _NW
    n_chunks = b_per_w // _CHUNK
    wid = lax.axis_index("s") * 2 + lax.axis_index("c")
    base = wid * b_per_w

    def body(i, carry):
        off = base + i * _CHUNK
        pltpu.sync_copy(idx_hbm.at[pl.ds(off, _CHUNK)], idx_v)
        pltpu.async_copy(table_hbm.at[idx_v], rows_v, sem).wait()
        pltpu.sync_copy(rows_v, out_hbm.at[pl.ds(off, _CHUNK)])
        return carry

    lax.fori_loop(0, n_chunks, body, 0)


def kernel(labels, table):
    flat = labels.reshape(-1).astype(jnp.int32)
    mesh = plsc.VectorSubcoreMesh(core_axis_name="c", subcore_axis_name="s")
    call = functools.partial(
        pl.kernel,
        mesh=mesh,
        out_type=jax.ShapeDtypeStruct((_BATCH * _FIELDS, _LATENT_DIM),
                                      jnp.float32),
        scratch_types=[
            pltpu.VMEM((_CHUNK,), jnp.int32),
            pltpu.VMEM((_CHUNK, _LATENT_DIM), jnp.float32),
            pltpu.SemaphoreType.DMA,
        ],
    )(_gather_kernel)
    out = call(flat, table)
    return out.reshape(_BATCH, _FIELDS, _LATENT_DIM)


# speedup vs baseline: 3.3711x; 1.0494x over previous
import functools

import jax
import jax.numpy as jnp
from jax import lax
from jax.experimental import pallas as pl
from jax.experimental.pallas import tpu as pltpu
from jax.experimental.pallas import tpu_sc as plsc

_NUM_CLASSES = 100000
_LATENT_DIM = 128
_BATCH = 16384
_FIELDS = 26

_NW = 32
_CHUNK = 416
_B_PER_W = (_BATCH * _FIELDS) // _NW
_N_CHUNKS = _B_PER_W // _CHUNK


def _gather_kernel(idx_hbm, table_hbm, out_hbm,
                   idx_v, rows0, rows1, gsem0, gsem1, wsem0, wsem1):
    rows = (rows0, rows1)
    gsem = (gsem0, gsem1)
    wsem = (wsem0, wsem1)
    wid = lax.axis_index("s") * 2 + lax.axis_index("c")
    base = wid * _B_PER_W

    pltpu.sync_copy(idx_hbm.at[pl.ds(base, _B_PER_W)], idx_v)

    def start_gather(i, b):
        pltpu.async_copy(
            table_hbm.at[idx_v.at[pl.ds(i * _CHUNK, _CHUNK)]], rows[b],
            gsem[b])

    def wait_gather(b):
        pltpu.make_async_copy(
            table_hbm.at[idx_v.at[pl.ds(0, _CHUNK)]], rows[b],
            gsem[b]).wait()

    def start_write(i, b):
        pltpu.async_copy(
            rows[b], out_hbm.at[pl.ds(base + i * _CHUNK, _CHUNK)], wsem[b])

    def wait_write(b):
        pltpu.make_async_copy(
            rows[b], out_hbm.at[pl.ds(base, _CHUNK)], wsem[b]).wait()

    start_gather(0, 0)
    wait_gather(0)
    start_write(0, 0)
    start_gather(1, 1)

    def body(g, carry):
        for b in range(2):
            i = 2 * g + b
            wait_gather(1 - b)
            start_write(i - 1, 1 - b)
            wait_write(b)
            start_gather(i, b)
        return carry

    lax.fori_loop(1, _N_CHUNKS // 2, body, 0)

    wait_gather(1)
    start_write(_N_CHUNKS - 1, 1)
    wait_write(0)
    wait_write(1)


def kernel(labels, table):
    flat = labels.reshape(-1).astype(jnp.int32)
    mesh = plsc.VectorSubcoreMesh(core_axis_name="c", subcore_axis_name="s")
    call = functools.partial(
        pl.kernel,
        mesh=mesh,
        out_type=jax.ShapeDtypeStruct((_BATCH * _FIELDS, _LATENT_DIM),
                                      jnp.float32),
        scratch_types=[
            pltpu.VMEM((_B_PER_W,), jnp.int32),
            pltpu.VMEM((_CHUNK, _LATENT_DIM), jnp.float32),
            pltpu.VMEM((_CHUNK, _LATENT_DIM), jnp.float32),
            pltpu.SemaphoreType.DMA,
            pltpu.SemaphoreType.DMA,
            pltpu.SemaphoreType.DMA,
            pltpu.SemaphoreType.DMA,
        ],
    )(_gather_kernel)
    out = call(flat, table)
    return out.reshape(_BATCH, _FIELDS, _LATENT_DIM)


# speedup vs baseline: 5.6327x; 1.6708x over previous
import functools

import jax
import jax.numpy as jnp
from jax import lax
from jax.experimental import pallas as pl
from jax.experimental.pallas import tpu as pltpu
from jax.experimental.pallas import tpu_sc as plsc

_NUM_CLASSES = 100000
_LATENT_DIM = 128
_BATCH = 16384
_FIELDS = 26

_NW = 32
_BPB = 16
_CHUNK = _BPB * _FIELDS
_B_PER_W = (_BATCH * _FIELDS) // _NW
_N_CHUNKS = _B_PER_W // _CHUNK


def _gather_kernel(idx_hbm, table_hbm, out_hbm,
                   idx_v, rows0, rows1, gsem0, gsem1, wsem0, wsem1):
    rows = (rows0, rows1)
    gsem = (gsem0, gsem1)
    wsem = (wsem0, wsem1)
    wid = lax.axis_index("s") * 2 + lax.axis_index("c")
    base = wid * _B_PER_W
    batch0 = wid * (_B_PER_W // _FIELDS)

    pltpu.sync_copy(idx_hbm.at[pl.ds(base, _B_PER_W)], idx_v)

    def start_gather(i, b):
        pltpu.async_copy(
            table_hbm.at[idx_v.at[pl.ds(i * _CHUNK, _CHUNK)]],
            rows[b], gsem[b])

    def wait_gather(b):
        pltpu.make_async_copy(
            table_hbm.at[idx_v.at[pl.ds(0, _CHUNK)]],
            rows[b], gsem[b]).wait()

    def start_write(i, b):
        for k in range(_BPB):
            pltpu.async_copy(
                rows[b].at[pl.ds(k * _FIELDS, _FIELDS)],
                out_hbm.at[batch0 + i * _BPB + k], wsem[b])

    def wait_write(b):
        pltpu.make_async_copy(
            table_hbm.at[pl.ds(0, _CHUNK)], rows[b], wsem[b]).wait()

    start_gather(0, 0)
    wait_gather(0)
    start_write(0, 0)
    start_gather(1, 1)

    def body(g, carry):
        for b in range(2):
            i = 2 * g + b
            wait_gather(1 - b)
            start_write(i - 1, 1 - b)
            wait_write(b)
            start_gather(i, b)
        return carry

    lax.fori_loop(1, _N_CHUNKS // 2, body, 0)

    wait_gather(1)
    start_write(_N_CHUNKS - 1, 1)
    wait_write(0)
    wait_write(1)


def kernel(labels, table):
    flat = labels.reshape(-1).astype(jnp.int32)
    mesh = plsc.VectorSubcoreMesh(core_axis_name="c", subcore_axis_name="s")
    call = functools.partial(
        pl.kernel,
        mesh=mesh,
        out_type=jax.ShapeDtypeStruct((_BATCH, _FIELDS, _LATENT_DIM),
                                      jnp.float32),
        compiler_params=pltpu.CompilerParams(use_tc_tiling_on_sc=True),
        scratch_types=[
            pltpu.VMEM((_B_PER_W,), jnp.int32),
            pltpu.VMEM((_CHUNK, _LATENT_DIM), jnp.float32),
            pltpu.VMEM((_CHUNK, _LATENT_DIM), jnp.float32),
            pltpu.SemaphoreType.DMA,
            pltpu.SemaphoreType.DMA,
            pltpu.SemaphoreType.DMA,
            pltpu.SemaphoreType.DMA,
        ],
    )(_gather_kernel)
    return call(flat, table)


# speedup vs baseline: 11.7007x; 2.0773x over previous
import functools

import jax
import jax.numpy as jnp
from jax import lax
from jax.experimental import pallas as pl
from jax.experimental.pallas import tpu as pltpu
from jax.experimental.pallas import tpu_sc as plsc

_NUM_CLASSES = 100000
_LATENT_DIM = 128
_BATCH = 16384
_FIELDS = 26

_NW = 32
_CHUNK = 416
_B_PER_W = (_BATCH * _FIELDS) // _NW
_N_CHUNKS = _B_PER_W // _CHUNK


def _gather_kernel(idx_hbm, table_hbm, out_hbm,
                   idx_v, rows0, rows1, gsem0, gsem1, wsem0, wsem1):
    rows = (rows0, rows1)
    gsem = (gsem0, gsem1)
    wsem = (wsem0, wsem1)
    wid = lax.axis_index("s") * 2 + lax.axis_index("c")
    base = wid * _B_PER_W

    pltpu.sync_copy(idx_hbm.at[pl.ds(base, _B_PER_W)], idx_v)

    def start_gather(i, b):
        pltpu.async_copy(
            table_hbm.at[idx_v.at[pl.ds(i * _CHUNK, _CHUNK)]], rows[b],
            gsem[b])

    def wait_gather(b):
        pltpu.make_async_copy(
            table_hbm.at[idx_v.at[pl.ds(0, _CHUNK)]], rows[b],
            gsem[b]).wait()

    def start_write(i, b):
        pltpu.async_copy(
            rows[b], out_hbm.at[pl.ds(base + i * _CHUNK, _CHUNK)], wsem[b])

    def wait_write(b):
        pltpu.make_async_copy(
            rows[b], out_hbm.at[pl.ds(base, _CHUNK)], wsem[b]).wait()

    start_gather(0, 0)
    wait_gather(0)
    start_write(0, 0)
    start_gather(1, 1)

    def body(g, carry):
        for b in range(2):
            i = 2 * g + b
            wait_gather(1 - b)
            start_write(i - 1, 1 - b)
            wait_write(b)
            start_gather(i, b)
        return carry

    lax.fori_loop(1, _N_CHUNKS // 2, body, 0)

    wait_gather(1)
    start_write(_N_CHUNKS - 1, 1)
    wait_write(0)
    wait_write(1)


def kernel(labels, table):
    flat = labels.astype(jnp.int32).T.reshape(-1)
    mesh = plsc.VectorSubcoreMesh(core_axis_name="c", subcore_axis_name="s")
    call = functools.partial(
        pl.kernel,
        mesh=mesh,
        out_type=jax.ShapeDtypeStruct((_BATCH * _FIELDS, _LATENT_DIM),
                                      jnp.float32),
        compiler_params=pltpu.CompilerParams(use_tc_tiling_on_sc=True),
        scratch_types=[
            pltpu.VMEM((_B_PER_W,), jnp.int32),
            pltpu.VMEM((_CHUNK, _LATENT_DIM), jnp.float32),
            pltpu.VMEM((_CHUNK, _LATENT_DIM), jnp.float32),
            pltpu.SemaphoreType.DMA,
            pltpu.SemaphoreType.DMA,
            pltpu.SemaphoreType.DMA,
            pltpu.SemaphoreType.DMA,
        ],
    )(_gather_kernel)
    out = call(flat, table)
    return out.reshape(_FIELDS, _BATCH, _LATENT_DIM).transpose(1, 0, 2)


# speedup vs baseline: 11.9269x; 1.0193x over previous
import functools

import jax
import jax.numpy as jnp
from jax import lax
from jax.experimental import pallas as pl
from jax.experimental.pallas import tpu as pltpu
from jax.experimental.pallas import tpu_sc as plsc

_NUM_CLASSES = 100000
_LATENT_DIM = 128
_BATCH = 16384
_FIELDS = 26

_NW = 32
_CHUNK = 208
_NBUF = 4
_B_PER_W = (_BATCH * _FIELDS) // _NW
_N_CHUNKS = _B_PER_W // _CHUNK


def _gather_kernel(idx_hbm, table_hbm, out_hbm,
                   idx_v, rows0, rows1, rows2, rows3,
                   gsem0, gsem1, gsem2, gsem3,
                   wsem0, wsem1, wsem2, wsem3):
    rows = (rows0, rows1, rows2, rows3)
    gsem = (gsem0, gsem1, gsem2, gsem3)
    wsem = (wsem0, wsem1, wsem2, wsem3)
    wid = lax.axis_index("s") * 2 + lax.axis_index("c")
    base = wid * _B_PER_W

    pltpu.sync_copy(idx_hbm.at[pl.ds(base, _B_PER_W)], idx_v)

    def start_gather(i, b):
        pltpu.async_copy(
            table_hbm.at[idx_v.at[pl.ds(i * _CHUNK, _CHUNK)]], rows[b],
            gsem[b])

    def wait_gather(b):
        pltpu.make_async_copy(
            table_hbm.at[idx_v.at[pl.ds(0, _CHUNK)]], rows[b],
            gsem[b]).wait()

    def start_write(i, b):
        pltpu.async_copy(
            rows[b], out_hbm.at[pl.ds(base + i * _CHUNK, _CHUNK)], wsem[b])

    def wait_write(b):
        pltpu.make_async_copy(
            rows[b], out_hbm.at[pl.ds(base, _CHUNK)], wsem[b]).wait()

    start_gather(0, 0)
    start_gather(1, 1)
    for j in (0, 1):
        wait_gather(j)
        start_write(j, j)
        start_gather(j + 2, j + 2)
    for j in (2, 3):
        wait_gather(j)
        start_write(j, j)
        wait_write(j - 2)
        start_gather(j + 2, j - 2)

    def body(g, carry):
        for k in range(_NBUF):
            j = _NBUF * (g + 1) + k
            wait_gather(k)
            start_write(j, k)
            wait_write((k + 2) % _NBUF)
            start_gather(j + 2, (k + 2) % _NBUF)
        return carry

    lax.fori_loop(0, (_N_CHUNKS - 8) // _NBUF, body, 0)

    for j in range(_N_CHUNKS - 4, _N_CHUNKS):
        k = j % _NBUF
        wait_gather(k)
        start_write(j, k)
        if j + 2 < _N_CHUNKS:
            wait_write((k + 2) % _NBUF)
            start_gather(j + 2, (k + 2) % _NBUF)
    for k in range(_NBUF):
        wait_write(k)


def kernel(labels, table):
    flat = labels.astype(jnp.int32).T.reshape(-1)
    mesh = plsc.VectorSubcoreMesh(core_axis_name="c", subcore_axis_name="s")
    call = functools.partial(
        pl.kernel,
        mesh=mesh,
        out_type=jax.ShapeDtypeStruct((_BATCH * _FIELDS, _LATENT_DIM),
                                      jnp.float32),
        compiler_params=pltpu.CompilerParams(use_tc_tiling_on_sc=True),
        scratch_types=[
            pltpu.VMEM((_B_PER_W,), jnp.int32),
            pltpu.VMEM((_CHUNK, _LATENT_DIM), jnp.float32),
            pltpu.VMEM((_CHUNK, _LATENT_DIM), jnp.float32),
            pltpu.VMEM((_CHUNK, _LATENT_DIM), jnp.float32),
            pltpu.VMEM((_CHUNK, _LATENT_DIM), jnp.float32),
            pltpu.SemaphoreType.DMA,
            pltpu.SemaphoreType.DMA,
            pltpu.SemaphoreType.DMA,
            pltpu.SemaphoreType.DMA,
            pltpu.SemaphoreType.DMA,
            pltpu.SemaphoreType.DMA,
            pltpu.SemaphoreType.DMA,
            pltpu.SemaphoreType.DMA,
        ],
    )(_gather_kernel)
    out = call(flat, table)
    return out.reshape(_FIELDS, _BATCH, _LATENT_DIM).transpose(1, 0, 2)
